# speedup vs baseline: 1.3524x; 1.3524x over previous
import functools

import jax
import jax.numpy as jnp
from jax import lax
from jax.experimental import pallas as pl
from jax.experimental.pallas import tpu as pltpu
from jax.experimental.pallas import tpu_sc as plsc

_NUM_CORES = 2
_NUM_SUBCORES = 16
_NW = _NUM_CORES * _NUM_SUBCORES
_IDX_MINOR = 128


@functools.cache
def _build(B, S, EMB):
  total = B * S
  chunk = total // _NW
  nsub = chunk // _IDX_MINOR
  assert chunk % _IDX_MINOR == 0 and S % chunk == 0

  mesh = plsc.VectorSubcoreMesh(core_axis_name="c", subcore_axis_name="s")

  @functools.partial(
      pl.kernel,
      out_type=jax.ShapeDtypeStruct((total, EMB), jnp.float32),
      mesh=mesh,
      scratch_types=[
          pltpu.VMEM((nsub, _IDX_MINOR), jnp.int32),
          pltpu.VMEM((chunk, EMB), jnp.float32),
          pltpu.SemaphoreType.DMA,
      ],
  )
  def emb_kernel(src_hbm, word_hbm, pos_hbm, out_hbm, idx_v, rows_v, sem):
    wid = lax.axis_index("s") * _NUM_CORES + lax.axis_index("c")
    base = wid * chunk
    pos_base = lax.rem(base, S)

    pltpu.sync_copy(src_hbm.at[wid], idx_v)
    pltpu.sync_copy(pos_hbm.at[pl.ds(pos_base, chunk)], rows_v)
    copies = [
        pltpu.async_copy(
            word_hbm.at[idx_v.at[j]],
            rows_v.at[pl.ds(j * _IDX_MINOR, _IDX_MINOR)],
            sem,
            add=True,
        )
        for j in range(nsub)
    ]
    for cp in copies:
      cp.wait()
    pltpu.sync_copy(rows_v, out_hbm.at[pl.ds(base, chunk)])

  return emb_kernel


def kernel(src, word_table, pos_table):
  B, S = src.shape
  EMB = word_table.shape[1]
  fn = _build(B, S, EMB)
  chunk = (B * S) // _NW
  src_r = src.astype(jnp.int32).reshape(_NW, chunk // _IDX_MINOR, _IDX_MINOR)
  out = fn(src_r, word_table, pos_table)
  return out.reshape(B, S, EMB)
